# Initial kernel scaffold; baseline (speedup 1.0000x reference)
import jax
import jax.numpy as jnp
from jax.experimental import pallas as pl


def kernel(pred_beta, pred_ccoords, pred_energy, pred_pos, pred_time, pred_id, t_idx, t_energy, t_pos, t_time):
    raise NotImplementedError("write your pallas kernel here")


# speedup vs baseline: 2.4376x; 2.4376x over previous
import functools

import jax
import jax.numpy as jnp
from jax.experimental import pallas as pl
from jax.experimental.pallas import tpu as pltpu

_N = 100000
_K = 256
_TILE = 2000
_T = _N // _TILE
_QMIN = 0.5
_SB = 1.0
_BIG = float(_N)


def _oc_body(feat_ref, out_ref, segc, betak, paynum, payden, amin, axa0, axa1,
             abeta, smem):
    ph = pl.program_id(0)
    t = pl.program_id(1)

    @pl.when((ph == 0) & (t == 0))
    def _init():
        segc[...] = jnp.zeros_like(segc)
        betak[...] = jnp.full_like(betak, -jnp.inf)
        paynum[...] = jnp.zeros_like(paynum)
        payden[...] = jnp.zeros_like(payden)
        amin[...] = jnp.full_like(amin, _BIG)
        axa0[...] = jnp.zeros_like(axa0)
        axa1[...] = jnp.zeros_like(axa1)
        abeta[...] = jnp.zeros_like(abeta)
        smem[0] = 0.0
        smem[1] = 0.0
        smem[2] = 0.0
        smem[3] = 0.0
        out_ref[...] = jnp.zeros_like(out_ref)

    feat = feat_ref[...]
    braw = feat[:, 0:1]
    x0 = feat[:, 1:2]
    x1 = feat[:, 2:3]
    pe = feat[:, 3:4]
    ppx = feat[:, 4:5]
    ppy = feat[:, 5:6]
    ptm = feat[:, 6:7]
    te = feat[:, 7:8]
    tpx = feat[:, 8:9]
    tpy = feat[:, 9:10]
    ttm = feat[:, 10:11]
    tidf = feat[:, 11:12]

    beta = jnp.clip(braw, 1e-6, 1.0 - 1e-6)
    noise = tidf == 0.0
    iotak = jax.lax.broadcasted_iota(jnp.int32, (1, _K), 1).astype(jnp.float32)
    onehot = tidf == iotak
    bmask = jnp.where(noise, -1.0, beta)

    @pl.when(ph == 0)
    def _p0():
        obj = jnp.where(onehot & (~noise), 1.0, 0.0)
        segc[...] += jnp.sum(obj, axis=0, keepdims=True)
        bk_t = jnp.max(jnp.where(onehot, bmask, -jnp.inf), axis=0,
                       keepdims=True)
        betak[...] = jnp.maximum(betak[...], bk_t)
        ew = jnp.maximum(
            jnp.where(te > 10.0, 1.0, (te - 0.5) / 10.0 * 10.0 / 9.5), 0.0)
        den = te + 1.0
        denz = den == 0.0
        le = jnp.where(denz, 0.0,
                       (te - pe) ** 2 / jnp.where(denz, 1.0, den))
        lpos = ((tpx - ppx) ** 2 + (tpy - ppy) ** 2) / 100.0
        lt = (ttm - ptm) ** 2
        pay = ew * le + lpos + lt
        pw = jnp.where(noise, 0.0, beta)
        paynum[...] += jnp.sum(jnp.where(onehot, pw * pay, 0.0), axis=0,
                               keepdims=True)
        payden[...] += jnp.sum(jnp.where(onehot, pw, 0.0), axis=0,
                               keepdims=True)
        smem[0] += jnp.sum(jnp.where(noise, beta, 0.0))
        smem[1] += jnp.sum(jnp.where(noise, 1.0, 0.0))

    @pl.when(ph == 1)
    def _p1():
        bkt = jnp.sum(jnp.where(onehot, betak[...], 0.0), axis=1,
                      keepdims=True)
        isal = onehot & (bmask == bkt) & (~noise)
        idxf = (jax.lax.broadcasted_iota(jnp.int32, (_TILE, 1), 0)
                .astype(jnp.float32) + jnp.float32(t) * _TILE)
        cand = jnp.where(isal, idxf, _BIG)
        tmin = jnp.min(cand, axis=0, keepdims=True)
        upd = tmin < amin[...]
        sel = cand == tmin
        px0 = jnp.sum(jnp.where(sel, x0, 0.0), axis=0, keepdims=True)
        px1 = jnp.sum(jnp.where(sel, x1, 0.0), axis=0, keepdims=True)
        pb = jnp.sum(jnp.where(sel, beta, 0.0), axis=0, keepdims=True)
        amin[...] = jnp.minimum(amin[...], tmin)
        axa0[...] = jnp.where(upd, px0, axa0[...])
        axa1[...] = jnp.where(upd, px1, axa1[...])
        abeta[...] = jnp.where(upd, pb, abeta[...])

    @pl.when(ph == 2)
    def _p2():
        ba = jnp.clip(abeta[...], 1e-6, 1.0 - 1e-6)
        qa = (0.5 * jnp.log((1.0 + ba) / (1.0 - ba))) ** 2 + _QMIN
        validf = jnp.where((segc[...] > 0.0) & (iotak > 0.0), 1.0, 0.0)
        q = (0.5 * jnp.log((1.0 + beta) / (1.0 - beta))) ** 2 + _QMIN
        d2 = (x0 - axa0[...]) ** 2 + (x1 - axa1[...]) ** 2
        dist = jnp.sqrt(d2 + 1e-6)
        w = qa * validf
        same = onehot & (~noise)
        att = jnp.where(same, d2, 0.0) * w * q
        rep = jnp.where(same, 0.0, jnp.maximum(1.0 - dist, 0.0)) * w * q
        smem[2] += jnp.sum(att)
        smem[3] += jnp.sum(rep)

        @pl.when(t == _T - 1)
        def _fin():
            validv = jnp.where((segc[...] > 0.0) & (iotak > 0.0), 1.0, 0.0)
            nv = jnp.sum(validv)
            att_den = jnp.sum(segc[...] * validv) + 1e-9
            v_att = smem[2] / att_den
            v_rep = smem[3] / (jnp.float32(_N) * nv + 1e-9)
            bkv = jnp.clip(betak[...], 0.0, 1.0)
            beta_obj = jnp.sum(jnp.where(validv > 0.0, 1.0 - bkv, 0.0)) / (
                nv + 1e-9)
            noise_l = _SB * smem[0] / (smem[1] + 1e-9)
            pdz = payden[...] == 0.0
            payk = jnp.where(pdz, 0.0,
                             paynum[...] / jnp.where(pdz, 1.0, payden[...]))
            pay_l = jnp.sum(validv * payk) / (nv + 1e-9)
            loss = (v_att + v_rep) + (beta_obj + noise_l) + pay_l
            out_ref[...] = loss * jnp.ones_like(out_ref)


@functools.partial(jax.jit, static_argnames=("interpret",))
def _oc_loss_pallas(feat, interpret=False):
    return pl.pallas_call(
        _oc_body,
        grid=(3, _T),
        in_specs=[pl.BlockSpec((_TILE, 16), lambda ph, t: (t, 0))],
        out_specs=pl.BlockSpec((1, 1), lambda ph, t: (0, 0)),
        out_shape=jax.ShapeDtypeStruct((1, 1), jnp.float32),
        scratch_shapes=[
            pltpu.VMEM((1, _K), jnp.float32),
            pltpu.VMEM((1, _K), jnp.float32),
            pltpu.VMEM((1, _K), jnp.float32),
            pltpu.VMEM((1, _K), jnp.float32),
            pltpu.VMEM((1, _K), jnp.float32),
            pltpu.VMEM((1, _K), jnp.float32),
            pltpu.VMEM((1, _K), jnp.float32),
            pltpu.VMEM((1, _K), jnp.float32),
            pltpu.SMEM((8,), jnp.float32),
        ],
        interpret=interpret,
    )(feat)


def kernel(pred_beta, pred_ccoords, pred_energy, pred_pos, pred_time,
           pred_id, t_idx, t_energy, t_pos, t_time):
    tidf = t_idx.reshape(-1, 1).astype(jnp.float32)
    feat = jnp.concatenate(
        [pred_beta, pred_ccoords, pred_energy, pred_pos, pred_time,
         t_energy, t_pos, t_time, tidf,
         jnp.zeros((_N, 4), jnp.float32)], axis=1)
    loss = _oc_loss_pallas(feat)
    return (pred_beta, loss.reshape(1))
